# baseline (speedup 1.0000x reference)
import functools

import jax
import jax.numpy as jnp
from jax import lax
from jax.experimental import pallas as pl
from jax.experimental.pallas import tpu as pltpu
from jax.experimental.pallas import tpu_sc as plsc

N = 10000
D_IN = 256
HID = 256
HEADS = 4
D_OUT = 256

NC = 2
NS = 16
NW = NC * NS

N_PAD = 10240
R_NODES = N_PAD // NS
E_TOT = 160000 + N
B_EDGE = 128
E_PAD = 172032
Q_EDGE = E_PAD // NS
NB = Q_EDGE // B_EDGE
BM = 1024
NM = N_PAD // BM

def _mesh():
    return plsc.VectorSubcoreMesh(
        core_axis_name="c", subcore_axis_name="s",
        num_cores=NC, num_subcores=NS)


def _tc1_body(x_ref, w_ref, avs_ref, avd_ref, hT_ref, asT_ref, adT_ref):
    c = pl.program_id(1)
    h = jnp.dot(x_ref[...], w_ref[...], preferred_element_type=jnp.float32)
    hT_ref[0] = h
    rows = lax.broadcasted_iota(jnp.int32, (8, 128), 0)
    a_s = jnp.sum(jnp.where(rows == c, avs_ref[...], 0.0), axis=0)
    a_d = jnp.sum(jnp.where(rows == c, avd_ref[...], 0.0), axis=0)
    pas = jnp.sum(h * a_s[None, :], axis=1)
    pad = jnp.sum(h * a_d[None, :], axis=1)
    head = c // 2
    rmask = lax.broadcasted_iota(jnp.int32, (8, BM), 0) == head
    vs = jnp.where(rmask, pas[None, :], 0.0)
    vd = jnp.where(rmask, pad[None, :], 0.0)

    @pl.when(c == 0)
    def _():
        asT_ref[...] = vs[:, None, :]
        adT_ref[...] = vd[:, None, :]

    @pl.when(c != 0)
    def _():
        asT_ref[...] += vs[:, None, :]
        adT_ref[...] += vd[:, None, :]


def _tc_layer1(x_pad, W1, avs, avd):
    return pl.pallas_call(
        _tc1_body,
        grid=(NM, 8),
        in_specs=[
            pl.BlockSpec((BM, D_IN), lambda i, c: (i, 0)),
            pl.BlockSpec((D_IN, 128), lambda i, c: (0, c)),
            pl.BlockSpec((8, 128), lambda i, c: (0, 0)),
            pl.BlockSpec((8, 128), lambda i, c: (0, 0)),
        ],
        out_specs=[
            pl.BlockSpec((1, BM, 128), lambda i, c: (c, i, 0)),
            pl.BlockSpec((8, 1, BM), lambda i, c: (0, 0, i)),
            pl.BlockSpec((8, 1, BM), lambda i, c: (0, 0, i)),
        ],
        out_shape=[
            jax.ShapeDtypeStruct((8, N_PAD, 128), jnp.float32),
            jax.ShapeDtypeStruct((8, 1, N_PAD), jnp.float32),
            jax.ShapeDtypeStruct((8, 1, N_PAD), jnp.float32),
        ],
    )(x_pad, W1, avs, avd)


def _tc2_body(agg_ref, b1_ref, w_ref, avs_ref, avd_ref, hT_ref, asT_ref, adT_ref):
    kc = pl.program_id(1)
    rows = lax.broadcasted_iota(jnp.int32, (8, 128), 0)
    b1c = jnp.sum(jnp.where(rows == kc, b1_ref[...], 0.0), axis=0)
    v = agg_ref[0] + b1c[None, :]
    xb = jnp.where(v > 0, v, jnp.exp(jnp.minimum(v, 0.0)) - 1.0)
    partial = jnp.dot(xb, w_ref[0], preferred_element_type=jnp.float32)

    @pl.when(kc == 0)
    def _():
        hT_ref[0] = partial[:, :128]
        hT_ref[1] = partial[:, 128:]

    @pl.when(kc != 0)
    def _():
        hT_ref[0] += partial[:, :128]
        hT_ref[1] += partial[:, 128:]

    @pl.when(kc == 7)
    def _():
        h0 = hT_ref[0]
        h1 = hT_ref[1]
        as2 = jnp.sum(h0 * avs_ref[0][None, :], axis=1) + jnp.sum(
            h1 * avs_ref[1][None, :], axis=1)
        ad2 = jnp.sum(h0 * avd_ref[0][None, :], axis=1) + jnp.sum(
            h1 * avd_ref[1][None, :], axis=1)
        rmask = lax.broadcasted_iota(jnp.int32, (8, BM), 0) == 0
        asT_ref[...] = jnp.where(rmask, as2[None, :], 0.0)[:, None, :]
        adT_ref[...] = jnp.where(rmask, ad2[None, :], 0.0)[:, None, :]


def _tc_layer2(agg1, b1r, W2r, avs2, avd2):
    return pl.pallas_call(
        _tc2_body,
        grid=(NM, 8),
        in_specs=[
            pl.BlockSpec((1, BM, 128), lambda i, kc: (kc, i, 0)),
            pl.BlockSpec((8, 128), lambda i, kc: (0, 0)),
            pl.BlockSpec((1, 128, D_OUT), lambda i, kc: (kc, 0, 0)),
            pl.BlockSpec((2, 128), lambda i, kc: (0, 0)),
            pl.BlockSpec((2, 128), lambda i, kc: (0, 0)),
        ],
        out_specs=[
            pl.BlockSpec((2, BM, 128), lambda i, kc: (0, i, 0)),
            pl.BlockSpec((8, 1, BM), lambda i, kc: (0, 0, i)),
            pl.BlockSpec((8, 1, BM), lambda i, kc: (0, 0, i)),
        ],
        out_shape=[
            jax.ShapeDtypeStruct((2, N_PAD, 128), jnp.float32),
            jax.ShapeDtypeStruct((8, 1, N_PAD), jnp.float32),
            jax.ShapeDtypeStruct((8, 1, N_PAD), jnp.float32),
        ],
    )(agg1, b1r, W2r, avs2, avd2)


def _coeff_body(hpc, n_heads, split_phase3, asT, adT, src, dst, alphaT,
                *refs):
    as_v = refs[:hpc]
    ad_v = refs[hpc:2 * hpc]
    s_v = refs[2 * hpc:3 * hpc]
    src_v, dst_v, alpha_v, acc_v, tmp_v, slots = refs[3 * hpc:]
    cid = lax.axis_index("c")
    sid = lax.axis_index("s")
    zero16 = jnp.zeros((16,), jnp.float32)

    for h in range(hpc):
        row = cid * hpc + h if n_heads > 1 else h
        pltpu.sync_copy(asT.at[row, 0], as_v[h])
        pltpu.sync_copy(adT.at[row, 0], ad_v[h])

    def zbody(j, _):
        for h in range(hpc):
            s_v[h][pl.ds(j * 16, 16)] = zero16
        return 0
    lax.fori_loop(0, N_PAD // 16, zbody, 0)

    lanes = lax.broadcasted_iota(jnp.int32, (16,), 0)
    dnums = lax.GatherDimensionNumbers(
        offset_dims=(), collapsed_slice_dims=(0,), start_index_map=(0,))
    gmax = []
    for h in range(hpc):
        def mbody(j, m, h=h):
            return jnp.maximum(m, as_v[h][pl.ds(j * 16, 16)])
        mv = lax.fori_loop(0, N_PAD // 16, mbody,
                           jnp.full((16,), -3.0e38, jnp.float32))
        for sh in (8, 4, 2, 1):
            perm = lax.gather(mv, jnp.bitwise_xor(lanes, sh)[:, None],
                              dnums, slice_sizes=(1,),
                              mode=lax.GatherScatterMode.PROMISE_IN_BOUNDS)
            mv = jnp.maximum(mv, perm)
        gmax.append(mv)

    ebase = sid * Q_EDGE
    pltpu.sync_copy(src.at[pl.ds(ebase, Q_EDGE)], src_v)
    pltpu.sync_copy(dst.at[pl.ds(ebase, Q_EDGE)], dst_v)

    def ebody(j, _):
        si = src_v[pl.ds(j * 16, 16)]
        di = dst_v[pl.ds(j * 16, 16)]
        for h in range(hpc):
            a_s = plsc.load_gather(as_v[h], [si])
            a_d = plsc.load_gather(ad_v[h], [di])
            e = a_s + a_d
            e = jnp.where(e > 0, e, 0.2 * e)
            mt = jnp.maximum(gmax[h] + a_d, 0.0)
            ex = jnp.exp(e - mt)
            plsc.addupdate_scatter(s_v[h], [di], ex)
        return 0
    lax.fori_loop(0, Q_EDGE // 16, ebody, 0)

    for h in range(hpc):
        pltpu.sync_copy(s_v[h], slots.at[sid, h])
    plsc.subcore_barrier()
    nbase = sid * R_NODES
    pltpu.sync_copy(slots.at[0, :, pl.ds(nbase, R_NODES)], acc_v)
    for t in range(1, NS):
        pltpu.sync_copy(slots.at[t, :, pl.ds(nbase, R_NODES)], tmp_v)

        def rbody(j, _):
            for h in range(hpc):
                acc_v[h, pl.ds(j * 16, 16)] += tmp_v[h, pl.ds(j * 16, 16)]
            return 0
        lax.fori_loop(0, R_NODES // 16, rbody, 0)
    plsc.subcore_barrier()
    pltpu.sync_copy(acc_v, slots.at[0, :, pl.ds(nbase, R_NODES)])
    plsc.subcore_barrier()
    for h in range(hpc):
        pltpu.sync_copy(slots.at[0, h], s_v[h])

    if split_phase3:
        q3 = E_PAD // (2 * NS)
        eb3 = cid * (E_PAD // 2) + sid * q3
        pltpu.sync_copy(src.at[pl.ds(eb3, q3)], src_v.at[pl.ds(0, q3)])
        pltpu.sync_copy(dst.at[pl.ds(eb3, q3)], dst_v.at[pl.ds(0, q3)])
    else:
        q3 = Q_EDGE
        eb3 = ebase

    for h in range(hpc):
        def abody(j, _, h=h):
            si = src_v[pl.ds(j * 16, 16)]
            di = dst_v[pl.ds(j * 16, 16)]
            a_s = plsc.load_gather(as_v[h], [si])
            a_d = plsc.load_gather(ad_v[h], [di])
            e = a_s + a_d
            e = jnp.where(e > 0, e, 0.2 * e)
            mt = jnp.maximum(gmax[h] + a_d, 0.0)
            ex = jnp.exp(e - mt)
            sv = plsc.load_gather(s_v[h], [di])
            alpha_v[pl.ds(j * 16, 16)] = ex / (sv + 1e-16)
            return 0
        lax.fori_loop(0, q3 // 16, abody, 0)
        row = cid * hpc + h if n_heads > 1 else h
        pltpu.sync_copy(alpha_v.at[pl.ds(0, q3)],
                        alphaT.at[row, 0, pl.ds(eb3, q3)])


def _sc_coeff(asT, adT, src, dst, n_heads):
    hpc = max(n_heads // NC, 1)
    split_phase3 = n_heads == 1
    body = functools.partial(_coeff_body, hpc, n_heads, split_phase3)
    f = pl.kernel(
        body,
        out_type=jax.ShapeDtypeStruct((n_heads, 1, E_PAD), jnp.float32),
        mesh=_mesh(),
        scratch_types=[
            *([pltpu.VMEM((N_PAD,), jnp.float32)] * (3 * hpc)),
            pltpu.VMEM((Q_EDGE,), jnp.int32),
            pltpu.VMEM((Q_EDGE,), jnp.int32),
            pltpu.VMEM((Q_EDGE,), jnp.float32),
            pltpu.VMEM((hpc, R_NODES), jnp.float32),
            pltpu.VMEM((hpc, R_NODES), jnp.float32),
            pltpu.VMEM_SHARED((NS, hpc, N_PAD), jnp.float32),
        ],
        compiler_params=pltpu.CompilerParams(needs_layout_passes=False),
    )
    return f(asT, adT, src, dst)


def _agg_body(cpc, n_heads, hT, alphaT, src2, dst2, zrows, out,
              src_v, dst_v, alpha_v, rows_v, gsem, isem, acc):
    cid = lax.axis_index("c")
    sid = lax.axis_index("s")
    ebase = sid * Q_EDGE
    rbase = sid * NB
    nbase = sid * R_NODES

    def trio(b, hrow):
        p = b % 2
        pltpu.async_copy(src2.at[rbase + b], src_v.at[p], isem.at[p])
        pltpu.async_copy(dst2.at[rbase + b], dst_v.at[p], isem.at[p])
        pltpu.async_copy(
            alphaT.at[hrow, 0, pl.ds(ebase + b * B_EDGE, B_EDGE)],
            alpha_v.at[p], isem.at[p])

    def trio_wait(b, hrow):
        p = b % 2
        pltpu.make_async_copy(src2.at[rbase + b], src_v.at[p],
                              isem.at[p]).wait()
        pltpu.make_async_copy(dst2.at[rbase + b], dst_v.at[p],
                              isem.at[p]).wait()
        pltpu.make_async_copy(
            alphaT.at[hrow, 0, pl.ds(ebase + b * B_EDGE, B_EDGE)],
            alpha_v.at[p], isem.at[p]).wait()

    for cc in range(cpc):
        g = cid * cpc + cc
        hrow = g // 2 if n_heads > 1 else 0
        pltpu.sync_copy(zrows, acc.at[pl.ds(nbase, R_NODES)])
        plsc.subcore_barrier()

        trio(0, hrow)
        trio(1, hrow)
        trio_wait(0, hrow)
        pltpu.async_copy(hT.at[g].at[src_v.at[0, 0]], rows_v.at[0],
                         gsem.at[0])

        def bbody(b, _):
            p = b % 2
            pltpu.make_async_copy(hT.at[g].at[src_v.at[p, 0]],
                                  rows_v.at[p], gsem.at[p]).wait()

            @pl.when(b + 1 < NB)
            def _():
                trio_wait(b + 1, hrow)
                pltpu.async_copy(hT.at[g].at[src_v.at[(b + 1) % 2, 0]],
                                 rows_v.at[(b + 1) % 2],
                                 gsem.at[(b + 1) % 2])

            def sbody(jj, _):
                av = alpha_v[p, pl.ds(jj * 16, 16)]
                for ii in range(16):
                    a = av[ii]
                    i = jj * 16 + ii
                    for r in range(8):
                        rows_v[p, i, pl.ds(r * 16, 16)] = (
                            rows_v[p, i, pl.ds(r * 16, 16)] * a)
                return 0
            lax.fori_loop(0, B_EDGE // 16, sbody, 0)
            pltpu.sync_copy(rows_v.at[p], acc.at[dst_v.at[p, 0]],
                            add=True)

            @pl.when(b + 2 < NB)
            def _():
                trio(b + 2, hrow)
            return 0
        lax.fori_loop(0, NB, bbody, 0)
        plsc.subcore_barrier()
        pltpu.sync_copy(acc.at[pl.ds(nbase, R_NODES)],
                        out.at[g].at[pl.ds(nbase, R_NODES)])


def _sc_agg(hT, alphaT, src2, dst2, zrows, n_chunks, n_heads):
    cpc = n_chunks // NC
    body = functools.partial(_agg_body, cpc, n_heads)
    f = pl.kernel(
        body,
        out_type=jax.ShapeDtypeStruct((n_chunks, N_PAD, 128), jnp.float32),
        mesh=_mesh(),
        scratch_types=[
            pltpu.VMEM((2, 1, B_EDGE), jnp.int32),
            pltpu.VMEM((2, 1, B_EDGE), jnp.int32),
            pltpu.VMEM((2, B_EDGE), jnp.float32),
            pltpu.VMEM((2, B_EDGE, 128), jnp.float32),
            pltpu.SemaphoreType.DMA((2,)),
            pltpu.SemaphoreType.DMA((2,)),
            pltpu.VMEM_SHARED((N_PAD, 128), jnp.float32),
        ],
    )
    return f(hT, alphaT, src2, dst2, zrows)


def kernel(x, edge_index, W1, a_src1, a_dst1, b1, W2, a_src2, a_dst2, b2):
    idx = edge_index.astype(jnp.int32)
    loop = jnp.arange(N, dtype=jnp.int32)
    src = jnp.concatenate(
        [idx[0], loop, jnp.zeros((E_PAD - E_TOT,), jnp.int32)])
    dst = jnp.concatenate(
        [idx[1], loop, jnp.full((E_PAD - E_TOT,), N, jnp.int32)])
    src2 = src.reshape(E_PAD // B_EDGE, 1, B_EDGE)
    dst2 = dst.reshape(E_PAD // B_EDGE, 1, B_EDGE)
    zrows = jnp.zeros((R_NODES, 128), jnp.float32)

    x_pad = jnp.pad(x, ((0, N_PAD - N), (0, 0)))
    avs1 = a_src1.reshape(8, 128)
    avd1 = a_dst1.reshape(8, 128)
    hT1, asT1, adT1 = _tc_layer1(x_pad, W1, avs1, avd1)
    alpha1 = _sc_coeff(asT1, adT1, src, dst, HEADS)
    agg1 = _sc_agg(hT1, alpha1, src2, dst2, zrows, 8, HEADS)

    b1r = b1.reshape(8, 128)
    W2r = W2.reshape(8, 128, D_OUT)
    avs2 = a_src2.reshape(2, 128)
    avd2 = a_dst2.reshape(2, 128)
    hT2, asT2, adT2 = _tc_layer2(agg1, b1r, W2r, avs2, avd2)
    alpha2 = _sc_coeff(asT2, adT2, src, dst, 1)
    agg2 = _sc_agg(hT2, alpha2, src2, dst2, zrows, 2, 1)

    out = jnp.concatenate([agg2[0, :N, :], agg2[1, :N, :]], axis=1)
    return out + b2[None, :]


# speedup vs baseline: 1.0023x; 1.0023x over previous
import functools

import jax
import jax.numpy as jnp
from jax import lax
from jax.experimental import pallas as pl
from jax.experimental.pallas import tpu as pltpu
from jax.experimental.pallas import tpu_sc as plsc

N = 10000
D_IN = 256
HID = 256
HEADS = 4
D_OUT = 256

NC = 2
NS = 16
NW = NC * NS

N_PAD = 10240
R_NODES = N_PAD // NS
E_TOT = 160000 + N
B_EDGE = 128
E_PAD = 172032
Q_EDGE = E_PAD // NS
NB = Q_EDGE // B_EDGE
BM = 1024
NM = N_PAD // BM

def _mesh():
    return plsc.VectorSubcoreMesh(
        core_axis_name="c", subcore_axis_name="s",
        num_cores=NC, num_subcores=NS)


def _tc1_body(x_ref, w_ref, avs_ref, avd_ref, hT_ref, asT_ref, adT_ref):
    c = pl.program_id(1)
    h = jnp.dot(x_ref[...], w_ref[...], preferred_element_type=jnp.float32)
    hT_ref[0] = h
    rows = lax.broadcasted_iota(jnp.int32, (8, 128), 0)
    a_s = jnp.sum(jnp.where(rows == c, avs_ref[...], 0.0), axis=0)
    a_d = jnp.sum(jnp.where(rows == c, avd_ref[...], 0.0), axis=0)
    pas = jnp.sum(h * a_s[None, :], axis=1)
    pad = jnp.sum(h * a_d[None, :], axis=1)
    head = c // 2
    rmask = lax.broadcasted_iota(jnp.int32, (8, BM), 0) == head
    vs = jnp.where(rmask, pas[None, :], 0.0)
    vd = jnp.where(rmask, pad[None, :], 0.0)

    @pl.when(c == 0)
    def _():
        asT_ref[...] = vs[:, None, :]
        adT_ref[...] = vd[:, None, :]

    @pl.when(c != 0)
    def _():
        asT_ref[...] += vs[:, None, :]
        adT_ref[...] += vd[:, None, :]


def _tc_layer1(x_pad, W1, avs, avd):
    return pl.pallas_call(
        _tc1_body,
        grid=(NM, 8),
        in_specs=[
            pl.BlockSpec((BM, D_IN), lambda i, c: (i, 0)),
            pl.BlockSpec((D_IN, 128), lambda i, c: (0, c)),
            pl.BlockSpec((8, 128), lambda i, c: (0, 0)),
            pl.BlockSpec((8, 128), lambda i, c: (0, 0)),
        ],
        out_specs=[
            pl.BlockSpec((1, BM, 128), lambda i, c: (c, i, 0)),
            pl.BlockSpec((8, 1, BM), lambda i, c: (0, 0, i)),
            pl.BlockSpec((8, 1, BM), lambda i, c: (0, 0, i)),
        ],
        out_shape=[
            jax.ShapeDtypeStruct((8, N_PAD, 128), jnp.float32),
            jax.ShapeDtypeStruct((8, 1, N_PAD), jnp.float32),
            jax.ShapeDtypeStruct((8, 1, N_PAD), jnp.float32),
        ],
    )(x_pad, W1, avs, avd)


def _tc2_body(agg_ref, b1_ref, w_ref, avs_ref, avd_ref, hT_ref, asT_ref, adT_ref):
    kc = pl.program_id(1)
    rows = lax.broadcasted_iota(jnp.int32, (8, 128), 0)
    b1c = jnp.sum(jnp.where(rows == kc, b1_ref[...], 0.0), axis=0)
    v = agg_ref[0] + b1c[None, :]
    xb = jnp.where(v > 0, v, jnp.exp(jnp.minimum(v, 0.0)) - 1.0)
    partial = jnp.dot(xb, w_ref[0], preferred_element_type=jnp.float32)

    @pl.when(kc == 0)
    def _():
        hT_ref[0] = partial[:, :128]
        hT_ref[1] = partial[:, 128:]

    @pl.when(kc != 0)
    def _():
        hT_ref[0] += partial[:, :128]
        hT_ref[1] += partial[:, 128:]

    @pl.when(kc == 7)
    def _():
        h0 = hT_ref[0]
        h1 = hT_ref[1]
        as2 = jnp.sum(h0 * avs_ref[0][None, :], axis=1) + jnp.sum(
            h1 * avs_ref[1][None, :], axis=1)
        ad2 = jnp.sum(h0 * avd_ref[0][None, :], axis=1) + jnp.sum(
            h1 * avd_ref[1][None, :], axis=1)
        rmask = lax.broadcasted_iota(jnp.int32, (8, BM), 0) == 0
        asT_ref[...] = jnp.where(rmask, as2[None, :], 0.0)[:, None, :]
        adT_ref[...] = jnp.where(rmask, ad2[None, :], 0.0)[:, None, :]


def _tc_layer2(agg1, b1r, W2r, avs2, avd2):
    return pl.pallas_call(
        _tc2_body,
        grid=(NM, 8),
        in_specs=[
            pl.BlockSpec((1, BM, 128), lambda i, kc: (kc, i, 0)),
            pl.BlockSpec((8, 128), lambda i, kc: (0, 0)),
            pl.BlockSpec((1, 128, D_OUT), lambda i, kc: (kc, 0, 0)),
            pl.BlockSpec((2, 128), lambda i, kc: (0, 0)),
            pl.BlockSpec((2, 128), lambda i, kc: (0, 0)),
        ],
        out_specs=[
            pl.BlockSpec((2, BM, 128), lambda i, kc: (0, i, 0)),
            pl.BlockSpec((8, 1, BM), lambda i, kc: (0, 0, i)),
            pl.BlockSpec((8, 1, BM), lambda i, kc: (0, 0, i)),
        ],
        out_shape=[
            jax.ShapeDtypeStruct((2, N_PAD, 128), jnp.float32),
            jax.ShapeDtypeStruct((8, 1, N_PAD), jnp.float32),
            jax.ShapeDtypeStruct((8, 1, N_PAD), jnp.float32),
        ],
    )(agg1, b1r, W2r, avs2, avd2)


def _coeff_body(hpc, n_heads, split_phase3, asT, adT, src, dst, alphaT,
                *refs):
    as_v = refs[:hpc]
    ad_v = refs[hpc:2 * hpc]
    s_v = refs[2 * hpc:3 * hpc]
    src_v, dst_v, alpha_v, acc_v, tmp_v, slots = refs[3 * hpc:]
    cid = lax.axis_index("c")
    sid = lax.axis_index("s")
    zero16 = jnp.zeros((16,), jnp.float32)

    for h in range(hpc):
        row = cid * hpc + h if n_heads > 1 else h
        pltpu.sync_copy(asT.at[row, 0], as_v[h])
        pltpu.sync_copy(adT.at[row, 0], ad_v[h])

    def zbody(j, _):
        for h in range(hpc):
            s_v[h][pl.ds(j * 16, 16)] = zero16
        return 0
    lax.fori_loop(0, N_PAD // 16, zbody, 0)

    lanes = lax.broadcasted_iota(jnp.int32, (16,), 0)
    dnums = lax.GatherDimensionNumbers(
        offset_dims=(), collapsed_slice_dims=(0,), start_index_map=(0,))
    gmax = []
    for h in range(hpc):
        def mbody(j, m, h=h):
            return jnp.maximum(m, as_v[h][pl.ds(j * 16, 16)])
        mv = lax.fori_loop(0, N_PAD // 16, mbody,
                           jnp.full((16,), -3.0e38, jnp.float32))
        for sh in (8, 4, 2, 1):
            perm = lax.gather(mv, jnp.bitwise_xor(lanes, sh)[:, None],
                              dnums, slice_sizes=(1,),
                              mode=lax.GatherScatterMode.PROMISE_IN_BOUNDS)
            mv = jnp.maximum(mv, perm)
        gmax.append(mv)

    ebase = sid * Q_EDGE
    pltpu.sync_copy(src.at[pl.ds(ebase, Q_EDGE)], src_v)
    pltpu.sync_copy(dst.at[pl.ds(ebase, Q_EDGE)], dst_v)

    def ebody(j, _):
        si = src_v[pl.ds(j * 16, 16)]
        di = dst_v[pl.ds(j * 16, 16)]
        for h in range(hpc):
            a_s = plsc.load_gather(as_v[h], [si])
            a_d = plsc.load_gather(ad_v[h], [di])
            e = a_s + a_d
            e = jnp.where(e > 0, e, 0.2 * e)
            mt = jnp.maximum(gmax[h] + a_d, 0.0)
            ex = jnp.exp(e - mt)
            plsc.addupdate_scatter(s_v[h], [di], ex)
        return 0
    lax.fori_loop(0, Q_EDGE // 16, ebody, 0)

    for h in range(hpc):
        pltpu.sync_copy(s_v[h], slots.at[sid, h])
    plsc.subcore_barrier()
    nbase = sid * R_NODES
    pltpu.sync_copy(slots.at[0, :, pl.ds(nbase, R_NODES)], acc_v)
    for t in range(1, NS):
        pltpu.sync_copy(slots.at[t, :, pl.ds(nbase, R_NODES)], tmp_v)

        def rbody(j, _):
            for h in range(hpc):
                acc_v[h, pl.ds(j * 16, 16)] += tmp_v[h, pl.ds(j * 16, 16)]
            return 0
        lax.fori_loop(0, R_NODES // 16, rbody, 0)
    plsc.subcore_barrier()
    pltpu.sync_copy(acc_v, slots.at[0, :, pl.ds(nbase, R_NODES)])
    plsc.subcore_barrier()
    for h in range(hpc):
        pltpu.sync_copy(slots.at[0, h], s_v[h])

    if split_phase3:
        q3 = E_PAD // (2 * NS)
        eb3 = cid * (E_PAD // 2) + sid * q3
        pltpu.sync_copy(src.at[pl.ds(eb3, q3)], src_v.at[pl.ds(0, q3)])
        pltpu.sync_copy(dst.at[pl.ds(eb3, q3)], dst_v.at[pl.ds(0, q3)])
    else:
        q3 = Q_EDGE
        eb3 = ebase

    for h in range(hpc):
        def abody(j, _, h=h):
            si = src_v[pl.ds(j * 16, 16)]
            di = dst_v[pl.ds(j * 16, 16)]
            a_s = plsc.load_gather(as_v[h], [si])
            a_d = plsc.load_gather(ad_v[h], [di])
            e = a_s + a_d
            e = jnp.where(e > 0, e, 0.2 * e)
            mt = jnp.maximum(gmax[h] + a_d, 0.0)
            ex = jnp.exp(e - mt)
            sv = plsc.load_gather(s_v[h], [di])
            alpha_v[pl.ds(j * 16, 16)] = ex / (sv + 1e-16)
            return 0
        lax.fori_loop(0, q3 // 16, abody, 0)
        row = cid * hpc + h if n_heads > 1 else h
        pltpu.sync_copy(alpha_v.at[pl.ds(0, q3)],
                        alphaT.at[row, 0, pl.ds(eb3, q3)])


def _sc_coeff(asT, adT, src, dst, n_heads):
    hpc = max(n_heads // NC, 1)
    split_phase3 = n_heads == 1
    body = functools.partial(_coeff_body, hpc, n_heads, split_phase3)
    f = pl.kernel(
        body,
        out_type=jax.ShapeDtypeStruct((n_heads, 1, E_PAD), jnp.float32),
        mesh=_mesh(),
        scratch_types=[
            *([pltpu.VMEM((N_PAD,), jnp.float32)] * (3 * hpc)),
            pltpu.VMEM((Q_EDGE,), jnp.int32),
            pltpu.VMEM((Q_EDGE,), jnp.int32),
            pltpu.VMEM((Q_EDGE,), jnp.float32),
            pltpu.VMEM((hpc, R_NODES), jnp.float32),
            pltpu.VMEM((hpc, R_NODES), jnp.float32),
            pltpu.VMEM_SHARED((NS, hpc, N_PAD), jnp.float32),
        ],
        compiler_params=pltpu.CompilerParams(needs_layout_passes=False),
    )
    return f(asT, adT, src, dst)


def _agg_body(cpc, n_heads, hT, alphaT, src2, dst2, zrows, out,
              src_v, dst_v, alpha_v, rows_v, gsem, isem, acc):
    cid = lax.axis_index("c")
    sid = lax.axis_index("s")
    ebase = sid * Q_EDGE
    rbase = sid * NB
    nbase = sid * R_NODES

    def trio(b, hrow):
        p = b % 2
        pltpu.async_copy(src2.at[rbase + b], src_v.at[p], isem.at[p])
        pltpu.async_copy(dst2.at[rbase + b], dst_v.at[p], isem.at[p])
        pltpu.async_copy(
            alphaT.at[hrow, 0, pl.ds(ebase + b * B_EDGE, B_EDGE)],
            alpha_v.at[p], isem.at[p])

    def trio_wait(b, hrow):
        p = b % 2
        pltpu.make_async_copy(src2.at[rbase + b], src_v.at[p],
                              isem.at[p]).wait()
        pltpu.make_async_copy(dst2.at[rbase + b], dst_v.at[p],
                              isem.at[p]).wait()
        pltpu.make_async_copy(
            alphaT.at[hrow, 0, pl.ds(ebase + b * B_EDGE, B_EDGE)],
            alpha_v.at[p], isem.at[p]).wait()

    for cc in range(cpc):
        g = cid * cpc + cc
        hrow = g // 2 if n_heads > 1 else 0
        pltpu.sync_copy(zrows, acc.at[pl.ds(nbase, R_NODES)])
        plsc.subcore_barrier()

        def scale_scatter(b):
            p = b % 2

            def sbody(jj, _):
                av = alpha_v[p, pl.ds(jj * 16, 16)]
                for ii in range(16):
                    a = av[ii]
                    i = jj * 16 + ii
                    for r in range(8):
                        rows_v[p, i, pl.ds(r * 16, 16)] = (
                            rows_v[p, i, pl.ds(r * 16, 16)] * a)
                return 0
            lax.fori_loop(0, B_EDGE // 16, sbody, 0)
            pltpu.sync_copy(rows_v.at[p], acc.at[dst_v.at[p, 0]],
                            add=True)

        def gwait(b):
            p = b % 2
            pltpu.make_async_copy(hT.at[g].at[src_v.at[p, 0]],
                                  rows_v.at[p], gsem.at[p]).wait()

        def gissue(b):
            p = b % 2
            pltpu.async_copy(hT.at[g].at[src_v.at[p, 0]], rows_v.at[p],
                             gsem.at[p])

        trio(0, hrow)
        trio(1, hrow)
        trio_wait(0, hrow)
        gissue(0)

        def bbody(b, _):
            gwait(b)
            trio_wait(b + 1, hrow)
            gissue(b + 1)
            scale_scatter(b)
            trio(b + 2, hrow)
            return 0
        lax.fori_loop(0, NB - 2, bbody, 0)
        gwait(NB - 2)
        trio_wait(NB - 1, hrow)
        gissue(NB - 1)
        scale_scatter(NB - 2)
        gwait(NB - 1)
        scale_scatter(NB - 1)
        plsc.subcore_barrier()
        pltpu.sync_copy(acc.at[pl.ds(nbase, R_NODES)],
                        out.at[g].at[pl.ds(nbase, R_NODES)])


def _sc_agg(hT, alphaT, src2, dst2, zrows, n_chunks, n_heads):
    cpc = n_chunks // NC
    body = functools.partial(_agg_body, cpc, n_heads)
    f = pl.kernel(
        body,
        out_type=jax.ShapeDtypeStruct((n_chunks, N_PAD, 128), jnp.float32),
        mesh=_mesh(),
        scratch_types=[
            pltpu.VMEM((2, 1, B_EDGE), jnp.int32),
            pltpu.VMEM((2, 1, B_EDGE), jnp.int32),
            pltpu.VMEM((2, B_EDGE), jnp.float32),
            pltpu.VMEM((2, B_EDGE, 128), jnp.float32),
            pltpu.SemaphoreType.DMA((2,)),
            pltpu.SemaphoreType.DMA((2,)),
            pltpu.VMEM_SHARED((N_PAD, 128), jnp.float32),
        ],
    )
    return f(hT, alphaT, src2, dst2, zrows)


def kernel(x, edge_index, W1, a_src1, a_dst1, b1, W2, a_src2, a_dst2, b2):
    idx = edge_index.astype(jnp.int32)
    loop = jnp.arange(N, dtype=jnp.int32)
    src = jnp.concatenate(
        [idx[0], loop, jnp.zeros((E_PAD - E_TOT,), jnp.int32)])
    dst = jnp.concatenate(
        [idx[1], loop, jnp.full((E_PAD - E_TOT,), N, jnp.int32)])
    src2 = src.reshape(E_PAD // B_EDGE, 1, B_EDGE)
    dst2 = dst.reshape(E_PAD // B_EDGE, 1, B_EDGE)
    zrows = jnp.zeros((R_NODES, 128), jnp.float32)

    x_pad = jnp.pad(x, ((0, N_PAD - N), (0, 0)))
    avs1 = a_src1.reshape(8, 128)
    avd1 = a_dst1.reshape(8, 128)
    hT1, asT1, adT1 = _tc_layer1(x_pad, W1, avs1, avd1)
    alpha1 = _sc_coeff(asT1, adT1, src, dst, HEADS)
    agg1 = _sc_agg(hT1, alpha1, src2, dst2, zrows, 8, HEADS)

    b1r = b1.reshape(8, 128)
    W2r = W2.reshape(8, 128, D_OUT)
    avs2 = a_src2.reshape(2, 128)
    avd2 = a_dst2.reshape(2, 128)
    hT2, asT2, adT2 = _tc_layer2(agg1, b1r, W2r, avs2, avd2)
    alpha2 = _sc_coeff(asT2, adT2, src, dst, 1)
    agg2 = _sc_agg(hT2, alpha2, src2, dst2, zrows, 2, 1)

    out = jnp.concatenate([agg2[0, :N, :], agg2[1, :N, :]], axis=1)
    return out + b2[None, :]


# speedup vs baseline: 1.0072x; 1.0049x over previous
import functools

import jax
import jax.numpy as jnp
from jax import lax
from jax.experimental import pallas as pl
from jax.experimental.pallas import tpu as pltpu
from jax.experimental.pallas import tpu_sc as plsc

N = 10000
D_IN = 256
HID = 256
HEADS = 4
D_OUT = 256

NC = 2
NS = 16
NW = NC * NS

N_PAD = 10240
R_NODES = N_PAD // NS
E_TOT = 160000 + N
B_EDGE = 128
E_PAD = 172032
Q_EDGE = E_PAD // NS
NB = Q_EDGE // B_EDGE
BM = 1024
NM = N_PAD // BM

def _mesh():
    return plsc.VectorSubcoreMesh(
        core_axis_name="c", subcore_axis_name="s",
        num_cores=NC, num_subcores=NS)


def _tc1_body(x_ref, w_ref, avs_ref, avd_ref, hT_ref, asT_ref, adT_ref):
    c = pl.program_id(1)
    h = jnp.dot(x_ref[...], w_ref[...], preferred_element_type=jnp.float32)
    hT_ref[0] = h
    rows = lax.broadcasted_iota(jnp.int32, (8, 128), 0)
    a_s = jnp.sum(jnp.where(rows == c, avs_ref[...], 0.0), axis=0)
    a_d = jnp.sum(jnp.where(rows == c, avd_ref[...], 0.0), axis=0)
    pas = jnp.sum(h * a_s[None, :], axis=1)
    pad = jnp.sum(h * a_d[None, :], axis=1)
    head = c // 2
    rmask = lax.broadcasted_iota(jnp.int32, (8, BM), 0) == head
    vs = jnp.where(rmask, pas[None, :], 0.0)
    vd = jnp.where(rmask, pad[None, :], 0.0)

    @pl.when(c == 0)
    def _():
        asT_ref[...] = vs[:, None, :]
        adT_ref[...] = vd[:, None, :]

    @pl.when(c != 0)
    def _():
        asT_ref[...] += vs[:, None, :]
        adT_ref[...] += vd[:, None, :]


def _tc_layer1(x_pad, W1, avs, avd):
    return pl.pallas_call(
        _tc1_body,
        grid=(NM, 8),
        in_specs=[
            pl.BlockSpec((BM, D_IN), lambda i, c: (i, 0)),
            pl.BlockSpec((D_IN, 128), lambda i, c: (0, c)),
            pl.BlockSpec((8, 128), lambda i, c: (0, 0)),
            pl.BlockSpec((8, 128), lambda i, c: (0, 0)),
        ],
        out_specs=[
            pl.BlockSpec((1, BM, 128), lambda i, c: (c, i, 0)),
            pl.BlockSpec((8, 1, BM), lambda i, c: (0, 0, i)),
            pl.BlockSpec((8, 1, BM), lambda i, c: (0, 0, i)),
        ],
        out_shape=[
            jax.ShapeDtypeStruct((8, N_PAD, 128), jnp.float32),
            jax.ShapeDtypeStruct((8, 1, N_PAD), jnp.float32),
            jax.ShapeDtypeStruct((8, 1, N_PAD), jnp.float32),
        ],
    )(x_pad, W1, avs, avd)


def _tc2_body(agg_ref, b1_ref, w_ref, avs_ref, avd_ref, hT_ref, asT_ref, adT_ref):
    kc = pl.program_id(1)
    rows = lax.broadcasted_iota(jnp.int32, (8, 128), 0)
    b1c = jnp.sum(jnp.where(rows == kc, b1_ref[...], 0.0), axis=0)
    v = agg_ref[0] + b1c[None, :]
    xb = jnp.where(v > 0, v, jnp.exp(jnp.minimum(v, 0.0)) - 1.0)
    partial = jnp.dot(xb, w_ref[0], preferred_element_type=jnp.float32)

    @pl.when(kc == 0)
    def _():
        hT_ref[0] = partial[:, :128]
        hT_ref[1] = partial[:, 128:]

    @pl.when(kc != 0)
    def _():
        hT_ref[0] += partial[:, :128]
        hT_ref[1] += partial[:, 128:]

    @pl.when(kc == 7)
    def _():
        h0 = hT_ref[0]
        h1 = hT_ref[1]
        as2 = jnp.sum(h0 * avs_ref[0][None, :], axis=1) + jnp.sum(
            h1 * avs_ref[1][None, :], axis=1)
        ad2 = jnp.sum(h0 * avd_ref[0][None, :], axis=1) + jnp.sum(
            h1 * avd_ref[1][None, :], axis=1)
        rmask = lax.broadcasted_iota(jnp.int32, (8, BM), 0) == 0
        asT_ref[...] = jnp.where(rmask, as2[None, :], 0.0)[:, None, :]
        adT_ref[...] = jnp.where(rmask, ad2[None, :], 0.0)[:, None, :]


def _tc_layer2(agg1, b1r, W2r, avs2, avd2):
    return pl.pallas_call(
        _tc2_body,
        grid=(NM, 8),
        in_specs=[
            pl.BlockSpec((1, BM, 128), lambda i, kc: (kc, i, 0)),
            pl.BlockSpec((8, 128), lambda i, kc: (0, 0)),
            pl.BlockSpec((1, 128, D_OUT), lambda i, kc: (kc, 0, 0)),
            pl.BlockSpec((2, 128), lambda i, kc: (0, 0)),
            pl.BlockSpec((2, 128), lambda i, kc: (0, 0)),
        ],
        out_specs=[
            pl.BlockSpec((2, BM, 128), lambda i, kc: (0, i, 0)),
            pl.BlockSpec((8, 1, BM), lambda i, kc: (0, 0, i)),
            pl.BlockSpec((8, 1, BM), lambda i, kc: (0, 0, i)),
        ],
        out_shape=[
            jax.ShapeDtypeStruct((2, N_PAD, 128), jnp.float32),
            jax.ShapeDtypeStruct((8, 1, N_PAD), jnp.float32),
            jax.ShapeDtypeStruct((8, 1, N_PAD), jnp.float32),
        ],
    )(agg1, b1r, W2r, avs2, avd2)


def _coeff_body(hpc, n_heads, split_phase3, asT, adT, src, dst, alphaT,
                *refs):
    as_v = refs[:hpc]
    ad_v = refs[hpc:2 * hpc]
    s_v = refs[2 * hpc:3 * hpc]
    src_v, dst_v, alpha_v, acc_v, tmp_v, slots = refs[3 * hpc:]
    cid = lax.axis_index("c")
    sid = lax.axis_index("s")
    zero16 = jnp.zeros((16,), jnp.float32)

    for h in range(hpc):
        row = cid * hpc + h if n_heads > 1 else h
        pltpu.sync_copy(asT.at[row, 0], as_v[h])
        pltpu.sync_copy(adT.at[row, 0], ad_v[h])

    def zbody(j, _):
        for h in range(hpc):
            s_v[h][pl.ds(j * 16, 16)] = zero16
        return 0
    lax.fori_loop(0, N_PAD // 16, zbody, 0)

    lanes = lax.broadcasted_iota(jnp.int32, (16,), 0)
    dnums = lax.GatherDimensionNumbers(
        offset_dims=(), collapsed_slice_dims=(0,), start_index_map=(0,))
    gmax = []
    for h in range(hpc):
        def mbody(j, m, h=h):
            return jnp.maximum(m, as_v[h][pl.ds(j * 16, 16)])
        mv = lax.fori_loop(0, N_PAD // 16, mbody,
                           jnp.full((16,), -3.0e38, jnp.float32))
        for sh in (8, 4, 2, 1):
            perm = lax.gather(mv, jnp.bitwise_xor(lanes, sh)[:, None],
                              dnums, slice_sizes=(1,),
                              mode=lax.GatherScatterMode.PROMISE_IN_BOUNDS)
            mv = jnp.maximum(mv, perm)
        gmax.append(mv)

    ebase = sid * Q_EDGE
    pltpu.sync_copy(src.at[pl.ds(ebase, Q_EDGE)], src_v)
    pltpu.sync_copy(dst.at[pl.ds(ebase, Q_EDGE)], dst_v)

    def ebody(j, _):
        si = src_v[pl.ds(j * 16, 16)]
        di = dst_v[pl.ds(j * 16, 16)]
        for h in range(hpc):
            a_s = plsc.load_gather(as_v[h], [si])
            a_d = plsc.load_gather(ad_v[h], [di])
            e = a_s + a_d
            e = jnp.where(e > 0, e, 0.2 * e)
            mt = jnp.maximum(gmax[h] + a_d, 0.0)
            ex = jnp.exp(e - mt)
            plsc.addupdate_scatter(s_v[h], [di], ex)
        return 0
    lax.fori_loop(0, Q_EDGE // 16, ebody, 0)

    for h in range(hpc):
        pltpu.sync_copy(s_v[h], slots.at[sid, h])
    plsc.subcore_barrier()
    nbase = sid * R_NODES
    pltpu.sync_copy(slots.at[0, :, pl.ds(nbase, R_NODES)], acc_v)
    for t in range(1, NS):
        pltpu.sync_copy(slots.at[t, :, pl.ds(nbase, R_NODES)], tmp_v)

        def rbody(j, _):
            for h in range(hpc):
                acc_v[h, pl.ds(j * 16, 16)] += tmp_v[h, pl.ds(j * 16, 16)]
            return 0
        lax.fori_loop(0, R_NODES // 16, rbody, 0)
    plsc.subcore_barrier()
    pltpu.sync_copy(acc_v, slots.at[0, :, pl.ds(nbase, R_NODES)])
    plsc.subcore_barrier()
    for h in range(hpc):
        pltpu.sync_copy(slots.at[0, h], s_v[h])

    if split_phase3:
        q3 = E_PAD // (2 * NS)
        eb3 = cid * (E_PAD // 2) + sid * q3
        pltpu.sync_copy(src.at[pl.ds(eb3, q3)], src_v.at[pl.ds(0, q3)])
        pltpu.sync_copy(dst.at[pl.ds(eb3, q3)], dst_v.at[pl.ds(0, q3)])
    else:
        q3 = Q_EDGE
        eb3 = ebase

    for h in range(hpc):
        def abody(j, _, h=h):
            si = src_v[pl.ds(j * 16, 16)]
            di = dst_v[pl.ds(j * 16, 16)]
            a_s = plsc.load_gather(as_v[h], [si])
            a_d = plsc.load_gather(ad_v[h], [di])
            e = a_s + a_d
            e = jnp.where(e > 0, e, 0.2 * e)
            mt = jnp.maximum(gmax[h] + a_d, 0.0)
            ex = jnp.exp(e - mt)
            sv = plsc.load_gather(s_v[h], [di])
            alpha_v[pl.ds(j * 16, 16)] = ex / (sv + 1e-16)
            return 0
        lax.fori_loop(0, q3 // 16, abody, 0)
        row = cid * hpc + h if n_heads > 1 else h
        pltpu.sync_copy(alpha_v.at[pl.ds(0, q3)],
                        alphaT.at[row, 0, pl.ds(eb3, q3)])


def _sc_coeff(asT, adT, src, dst, n_heads):
    hpc = max(n_heads // NC, 1)
    split_phase3 = n_heads == 1
    body = functools.partial(_coeff_body, hpc, n_heads, split_phase3)
    f = pl.kernel(
        body,
        out_type=jax.ShapeDtypeStruct((n_heads, 1, E_PAD), jnp.float32),
        mesh=_mesh(),
        scratch_types=[
            *([pltpu.VMEM((N_PAD,), jnp.float32)] * (3 * hpc)),
            pltpu.VMEM((Q_EDGE,), jnp.int32),
            pltpu.VMEM((Q_EDGE,), jnp.int32),
            pltpu.VMEM((Q_EDGE,), jnp.float32),
            pltpu.VMEM((hpc, R_NODES), jnp.float32),
            pltpu.VMEM((hpc, R_NODES), jnp.float32),
            pltpu.VMEM_SHARED((NS, hpc, N_PAD), jnp.float32),
        ],
        compiler_params=pltpu.CompilerParams(needs_layout_passes=False),
    )
    return f(asT, adT, src, dst)


def _agg_body(cpc, n_heads, hT, alphaT, sd2, zrows, out,
              sd_v, alpha_v, rows_v, gsem, isem, acc):
    cid = lax.axis_index("c")
    sid = lax.axis_index("s")
    ebase = sid * Q_EDGE
    rbase = sid * NB
    nbase = sid * R_NODES

    def sd_issue(b):
        pltpu.async_copy(sd2.at[rbase + b], sd_v.at[b % 2], isem.at[b % 2])

    def sd_wait(b):
        pltpu.make_async_copy(sd2.at[rbase + b], sd_v.at[b % 2],
                              isem.at[b % 2]).wait()

    for cc in range(cpc):
        g = cid * cpc + cc
        hrow = g // 2 if n_heads > 1 else 0
        pltpu.sync_copy(alphaT.at[hrow, 0, pl.ds(ebase, Q_EDGE)], alpha_v)
        pltpu.sync_copy(zrows, acc.at[pl.ds(nbase, R_NODES)])
        plsc.subcore_barrier()

        def scale_scatter(b):
            p = b % 2

            def sbody(jj, _):
                av = alpha_v[pl.ds(b * B_EDGE + jj * 16, 16)]
                for ii in range(16):
                    a = av[ii]
                    i = jj * 16 + ii
                    for r in range(8):
                        rows_v[p, i, pl.ds(r * 16, 16)] = (
                            rows_v[p, i, pl.ds(r * 16, 16)] * a)
                return 0
            lax.fori_loop(0, B_EDGE // 16, sbody, 0)
            pltpu.sync_copy(rows_v.at[p], acc.at[sd_v.at[p, 1]],
                            add=True)

        def gwait(b):
            p = b % 2
            pltpu.make_async_copy(hT.at[g].at[sd_v.at[p, 0]],
                                  rows_v.at[p], gsem.at[p]).wait()

        def gissue(b):
            p = b % 2
            pltpu.async_copy(hT.at[g].at[sd_v.at[p, 0]], rows_v.at[p],
                             gsem.at[p])

        sd_issue(0)
        sd_issue(1)
        sd_wait(0)
        gissue(0)

        def bbody(b, _):
            gwait(b)
            sd_wait(b + 1)
            gissue(b + 1)
            scale_scatter(b)
            sd_issue(b + 2)
            return 0
        lax.fori_loop(0, NB - 2, bbody, 0)
        gwait(NB - 2)
        sd_wait(NB - 1)
        gissue(NB - 1)
        scale_scatter(NB - 2)
        gwait(NB - 1)
        scale_scatter(NB - 1)
        plsc.subcore_barrier()
        pltpu.sync_copy(acc.at[pl.ds(nbase, R_NODES)],
                        out.at[g].at[pl.ds(nbase, R_NODES)])


def _sc_agg(hT, alphaT, sd2, zrows, n_chunks, n_heads):
    cpc = n_chunks // NC
    body = functools.partial(_agg_body, cpc, n_heads)
    f = pl.kernel(
        body,
        out_type=jax.ShapeDtypeStruct((n_chunks, N_PAD, 128), jnp.float32),
        mesh=_mesh(),
        scratch_types=[
            pltpu.VMEM((2, 2, B_EDGE), jnp.int32),
            pltpu.VMEM((Q_EDGE,), jnp.float32),
            pltpu.VMEM((2, B_EDGE, 128), jnp.float32),
            pltpu.SemaphoreType.DMA((2,)),
            pltpu.SemaphoreType.DMA((2,)),
            pltpu.VMEM_SHARED((N_PAD, 128), jnp.float32),
        ],
    )
    return f(hT, alphaT, sd2, zrows)


def kernel(x, edge_index, W1, a_src1, a_dst1, b1, W2, a_src2, a_dst2, b2):
    idx = edge_index.astype(jnp.int32)
    loop = jnp.arange(N, dtype=jnp.int32)
    src = jnp.concatenate(
        [idx[0], loop, jnp.zeros((E_PAD - E_TOT,), jnp.int32)])
    dst = jnp.concatenate(
        [idx[1], loop, jnp.full((E_PAD - E_TOT,), N, jnp.int32)])
    sd2 = jnp.stack([src.reshape(E_PAD // B_EDGE, B_EDGE),
                     dst.reshape(E_PAD // B_EDGE, B_EDGE)], axis=1)
    zrows = jnp.zeros((R_NODES, 128), jnp.float32)

    x_pad = jnp.pad(x, ((0, N_PAD - N), (0, 0)))
    avs1 = a_src1.reshape(8, 128)
    avd1 = a_dst1.reshape(8, 128)
    hT1, asT1, adT1 = _tc_layer1(x_pad, W1, avs1, avd1)
    alpha1 = _sc_coeff(asT1, adT1, src, dst, HEADS)
    agg1 = _sc_agg(hT1, alpha1, sd2, zrows, 8, HEADS)

    b1r = b1.reshape(8, 128)
    W2r = W2.reshape(8, 128, D_OUT)
    avs2 = a_src2.reshape(2, 128)
    avd2 = a_dst2.reshape(2, 128)
    hT2, asT2, adT2 = _tc_layer2(agg1, b1r, W2r, avs2, avd2)
    alpha2 = _sc_coeff(asT2, adT2, src, dst, 1)
    agg2 = _sc_agg(hT2, alpha2, sd2, zrows, 2, 1)

    out = jnp.concatenate([agg2[0, :N, :], agg2[1, :N, :]], axis=1)
    return out + b2[None, :]


# speedup vs baseline: 1.5235x; 1.5126x over previous
import functools

import jax
import jax.numpy as jnp
from jax import lax
from jax.experimental import pallas as pl
from jax.experimental.pallas import tpu as pltpu
from jax.experimental.pallas import tpu_sc as plsc

N = 10000
D_IN = 256
HID = 256
HEADS = 4
D_OUT = 256

NC = 2
NS = 16
NW = NC * NS

N_PAD = 10240
R_NODES = N_PAD // NS
E_TOT = 160000 + N
B_EDGE = 128
E_PAD = 172032
Q_EDGE = E_PAD // NS
NB = Q_EDGE // B_EDGE
BM = 1024
NM = N_PAD // BM

def _mesh():
    return plsc.VectorSubcoreMesh(
        core_axis_name="c", subcore_axis_name="s",
        num_cores=NC, num_subcores=NS)


def _tc1_body(x_ref, w_ref, avs_ref, avd_ref, hT_ref, asT_ref, adT_ref):
    c = pl.program_id(1)
    h = jnp.dot(x_ref[...], w_ref[...], preferred_element_type=jnp.float32)
    hT_ref[0] = h
    rows = lax.broadcasted_iota(jnp.int32, (8, 128), 0)
    a_s = jnp.sum(jnp.where(rows == c, avs_ref[...], 0.0), axis=0)
    a_d = jnp.sum(jnp.where(rows == c, avd_ref[...], 0.0), axis=0)
    pas = jnp.sum(h * a_s[None, :], axis=1)
    pad = jnp.sum(h * a_d[None, :], axis=1)
    head = c // 2
    rmask = lax.broadcasted_iota(jnp.int32, (8, BM), 0) == head
    vs = jnp.where(rmask, pas[None, :], 0.0)
    vd = jnp.where(rmask, pad[None, :], 0.0)

    @pl.when(c == 0)
    def _():
        asT_ref[...] = vs[:, None, :]
        adT_ref[...] = vd[:, None, :]

    @pl.when(c != 0)
    def _():
        asT_ref[...] += vs[:, None, :]
        adT_ref[...] += vd[:, None, :]


def _tc_layer1(x_pad, W1, avs, avd):
    return pl.pallas_call(
        _tc1_body,
        grid=(NM, 8),
        in_specs=[
            pl.BlockSpec((BM, D_IN), lambda i, c: (i, 0)),
            pl.BlockSpec((D_IN, 128), lambda i, c: (0, c)),
            pl.BlockSpec((8, 128), lambda i, c: (0, 0)),
            pl.BlockSpec((8, 128), lambda i, c: (0, 0)),
        ],
        out_specs=[
            pl.BlockSpec((1, BM, 128), lambda i, c: (c, i, 0)),
            pl.BlockSpec((8, 1, BM), lambda i, c: (0, 0, i)),
            pl.BlockSpec((8, 1, BM), lambda i, c: (0, 0, i)),
        ],
        out_shape=[
            jax.ShapeDtypeStruct((8, N_PAD, 128), jnp.float32),
            jax.ShapeDtypeStruct((8, 1, N_PAD), jnp.float32),
            jax.ShapeDtypeStruct((8, 1, N_PAD), jnp.float32),
        ],
    )(x_pad, W1, avs, avd)


def _tc2_body(agg_ref, b1_ref, w_ref, avs_ref, avd_ref, hT_ref, asT_ref, adT_ref):
    kc = pl.program_id(1)
    rows = lax.broadcasted_iota(jnp.int32, (8, 128), 0)
    b1c = jnp.sum(jnp.where(rows == kc, b1_ref[...], 0.0), axis=0)
    v = agg_ref[0] + b1c[None, :]
    xb = jnp.where(v > 0, v, jnp.exp(jnp.minimum(v, 0.0)) - 1.0)
    partial = jnp.dot(xb, w_ref[0], preferred_element_type=jnp.float32)

    @pl.when(kc == 0)
    def _():
        hT_ref[0] = partial[:, :128]
        hT_ref[1] = partial[:, 128:]

    @pl.when(kc != 0)
    def _():
        hT_ref[0] += partial[:, :128]
        hT_ref[1] += partial[:, 128:]

    @pl.when(kc == 7)
    def _():
        h0 = hT_ref[0]
        h1 = hT_ref[1]
        as2 = jnp.sum(h0 * avs_ref[0][None, :], axis=1) + jnp.sum(
            h1 * avs_ref[1][None, :], axis=1)
        ad2 = jnp.sum(h0 * avd_ref[0][None, :], axis=1) + jnp.sum(
            h1 * avd_ref[1][None, :], axis=1)
        rmask = lax.broadcasted_iota(jnp.int32, (8, BM), 0) == 0
        asT_ref[...] = jnp.where(rmask, as2[None, :], 0.0)[:, None, :]
        adT_ref[...] = jnp.where(rmask, ad2[None, :], 0.0)[:, None, :]


def _tc_layer2(agg1, b1r, W2r, avs2, avd2):
    return pl.pallas_call(
        _tc2_body,
        grid=(NM, 8),
        in_specs=[
            pl.BlockSpec((1, BM, 128), lambda i, kc: (kc, i, 0)),
            pl.BlockSpec((8, 128), lambda i, kc: (0, 0)),
            pl.BlockSpec((1, 128, D_OUT), lambda i, kc: (kc, 0, 0)),
            pl.BlockSpec((2, 128), lambda i, kc: (0, 0)),
            pl.BlockSpec((2, 128), lambda i, kc: (0, 0)),
        ],
        out_specs=[
            pl.BlockSpec((2, BM, 128), lambda i, kc: (0, i, 0)),
            pl.BlockSpec((8, 1, BM), lambda i, kc: (0, 0, i)),
            pl.BlockSpec((8, 1, BM), lambda i, kc: (0, 0, i)),
        ],
        out_shape=[
            jax.ShapeDtypeStruct((2, N_PAD, 128), jnp.float32),
            jax.ShapeDtypeStruct((8, 1, N_PAD), jnp.float32),
            jax.ShapeDtypeStruct((8, 1, N_PAD), jnp.float32),
        ],
    )(agg1, b1r, W2r, avs2, avd2)


def _coeff_body(hpc, n_heads, split_phase3, asT, adT, src, dst, alphaT,
                *refs):
    as_v = refs[:hpc]
    ad_v = refs[hpc:2 * hpc]
    s_v = refs[2 * hpc:3 * hpc]
    src_v, dst_v, alpha_v, acc_v, tmp_v, slots = refs[3 * hpc:]
    cid = lax.axis_index("c")
    sid = lax.axis_index("s")
    zero16 = jnp.zeros((16,), jnp.float32)

    for h in range(hpc):
        row = cid * hpc + h if n_heads > 1 else h
        pltpu.sync_copy(asT.at[row, 0], as_v[h])
        pltpu.sync_copy(adT.at[row, 0], ad_v[h])

    def zbody(j, _):
        for h in range(hpc):
            s_v[h][pl.ds(j * 16, 16)] = zero16
        return 0
    lax.fori_loop(0, N_PAD // 16, zbody, 0)

    lanes = lax.broadcasted_iota(jnp.int32, (16,), 0)
    dnums = lax.GatherDimensionNumbers(
        offset_dims=(), collapsed_slice_dims=(0,), start_index_map=(0,))
    gmax = []
    for h in range(hpc):
        def mbody(j, m, h=h):
            return jnp.maximum(m, as_v[h][pl.ds(j * 16, 16)])
        mv = lax.fori_loop(0, N_PAD // 16, mbody,
                           jnp.full((16,), -3.0e38, jnp.float32))
        for sh in (8, 4, 2, 1):
            perm = lax.gather(mv, jnp.bitwise_xor(lanes, sh)[:, None],
                              dnums, slice_sizes=(1,),
                              mode=lax.GatherScatterMode.PROMISE_IN_BOUNDS)
            mv = jnp.maximum(mv, perm)
        gmax.append(mv)

    ebase = sid * Q_EDGE
    pltpu.sync_copy(src.at[pl.ds(ebase, Q_EDGE)], src_v)
    pltpu.sync_copy(dst.at[pl.ds(ebase, Q_EDGE)], dst_v)

    def ebody(j, _):
        si = src_v[pl.ds(j * 16, 16)]
        di = dst_v[pl.ds(j * 16, 16)]
        for h in range(hpc):
            a_s = plsc.load_gather(as_v[h], [si])
            a_d = plsc.load_gather(ad_v[h], [di])
            e = a_s + a_d
            e = jnp.where(e > 0, e, 0.2 * e)
            mt = jnp.maximum(gmax[h] + a_d, 0.0)
            ex = jnp.exp(e - mt)
            plsc.addupdate_scatter(s_v[h], [di], ex)
        return 0
    lax.fori_loop(0, Q_EDGE // 16, ebody, 0)

    for h in range(hpc):
        pltpu.sync_copy(s_v[h], slots.at[sid, h])
    plsc.subcore_barrier()
    nbase = sid * R_NODES
    pltpu.sync_copy(slots.at[0, :, pl.ds(nbase, R_NODES)], acc_v)
    for t in range(1, NS):
        pltpu.sync_copy(slots.at[t, :, pl.ds(nbase, R_NODES)], tmp_v)

        def rbody(j, _):
            for h in range(hpc):
                acc_v[h, pl.ds(j * 16, 16)] += tmp_v[h, pl.ds(j * 16, 16)]
            return 0
        lax.fori_loop(0, R_NODES // 16, rbody, 0)
    plsc.subcore_barrier()
    pltpu.sync_copy(acc_v, slots.at[0, :, pl.ds(nbase, R_NODES)])
    plsc.subcore_barrier()
    for h in range(hpc):
        pltpu.sync_copy(slots.at[0, h], s_v[h])

    if split_phase3:
        q3 = E_PAD // (2 * NS)
        eb3 = cid * (E_PAD // 2) + sid * q3
        pltpu.sync_copy(src.at[pl.ds(eb3, q3)], src_v.at[pl.ds(0, q3)])
        pltpu.sync_copy(dst.at[pl.ds(eb3, q3)], dst_v.at[pl.ds(0, q3)])
    else:
        q3 = Q_EDGE
        eb3 = ebase

    for h in range(hpc):
        def abody(j, _, h=h):
            si = src_v[pl.ds(j * 16, 16)]
            di = dst_v[pl.ds(j * 16, 16)]
            a_s = plsc.load_gather(as_v[h], [si])
            a_d = plsc.load_gather(ad_v[h], [di])
            e = a_s + a_d
            e = jnp.where(e > 0, e, 0.2 * e)
            mt = jnp.maximum(gmax[h] + a_d, 0.0)
            ex = jnp.exp(e - mt)
            sv = plsc.load_gather(s_v[h], [di])
            alpha_v[pl.ds(j * 16, 16)] = ex / (sv + 1e-16)
            return 0
        lax.fori_loop(0, q3 // 16, abody, 0)
        row = cid * hpc + h if n_heads > 1 else h
        pltpu.sync_copy(alpha_v.at[pl.ds(0, q3)],
                        alphaT.at[row, 0, pl.ds(eb3, q3)])


def _sc_coeff(asT, adT, src, dst, n_heads):
    hpc = max(n_heads // NC, 1)
    split_phase3 = n_heads == 1
    body = functools.partial(_coeff_body, hpc, n_heads, split_phase3)
    f = pl.kernel(
        body,
        out_type=jax.ShapeDtypeStruct((n_heads, 1, E_PAD), jnp.float32),
        mesh=_mesh(),
        scratch_types=[
            *([pltpu.VMEM((N_PAD,), jnp.float32)] * (3 * hpc)),
            pltpu.VMEM((Q_EDGE,), jnp.int32),
            pltpu.VMEM((Q_EDGE,), jnp.int32),
            pltpu.VMEM((Q_EDGE,), jnp.float32),
            pltpu.VMEM((hpc, R_NODES), jnp.float32),
            pltpu.VMEM((hpc, R_NODES), jnp.float32),
            pltpu.VMEM_SHARED((NS, hpc, N_PAD), jnp.float32),
        ],
        compiler_params=pltpu.CompilerParams(needs_layout_passes=False),
    )
    return f(asT, adT, src, dst)


def _agg_body(cpc, n_heads, hT, alphaT, sd2, zrows, out,
              sd_v, alpha_v, rows_v, gsem, isem, acc):
    cid = lax.axis_index("c")
    sid = lax.axis_index("s")
    ebase = sid * Q_EDGE
    rbase = sid * NB
    nbase = sid * R_NODES

    pltpu.sync_copy(sd2.at[pl.ds(rbase, NB)], sd_v)
    for cc in range(cpc):
        g = cid * cpc + cc
        hrow = g // 2 if n_heads > 1 else 0
        pltpu.sync_copy(alphaT.at[hrow, 0, pl.ds(ebase, Q_EDGE)], alpha_v)
        pltpu.sync_copy(zrows, acc.at[pl.ds(nbase, R_NODES)])
        plsc.subcore_barrier()

        def bbody(b, _):
            pltpu.async_copy(hT.at[g].at[sd_v.at[b, 0]], rows_v,
                             gsem).wait()

            def sbody(jj, _):
                av = alpha_v[pl.ds(b * B_EDGE + jj * 16, 16)]
                for ii in range(16):
                    a = av[ii]
                    i = jj * 16 + ii
                    for r in range(8):
                        rows_v[i, pl.ds(r * 16, 16)] = (
                            rows_v[i, pl.ds(r * 16, 16)] * a)
                return 0
            lax.fori_loop(0, B_EDGE // 16, sbody, 0)
            pltpu.sync_copy(rows_v, acc.at[sd_v.at[b, 1]], add=True)
            return 0
        lax.fori_loop(0, NB, bbody, 0)
        plsc.subcore_barrier()
        pltpu.sync_copy(acc.at[pl.ds(nbase, R_NODES)],
                        out.at[g].at[pl.ds(nbase, R_NODES)])


def _sc_agg(hT, alphaT, sd2, zrows, n_chunks, n_heads):
    cpc = n_chunks // NC
    body = functools.partial(_agg_body, cpc, n_heads)
    f = pl.kernel(
        body,
        out_type=jax.ShapeDtypeStruct((n_chunks, N_PAD, 128), jnp.float32),
        mesh=_mesh(),
        scratch_types=[
            pltpu.VMEM((NB, 2, B_EDGE), jnp.int32),
            pltpu.VMEM((Q_EDGE,), jnp.float32),
            pltpu.VMEM((B_EDGE, 128), jnp.float32),
            pltpu.SemaphoreType.DMA,
            pltpu.SemaphoreType.DMA,
            pltpu.VMEM_SHARED((N_PAD, 128), jnp.float32),
        ],
    )
    return f(hT, alphaT, sd2, zrows)


def kernel(x, edge_index, W1, a_src1, a_dst1, b1, W2, a_src2, a_dst2, b2):
    idx = edge_index.astype(jnp.int32)
    loop = jnp.arange(N, dtype=jnp.int32)
    src = jnp.concatenate(
        [idx[0], loop, jnp.zeros((E_PAD - E_TOT,), jnp.int32)])
    dst = jnp.concatenate(
        [idx[1], loop, jnp.full((E_PAD - E_TOT,), N, jnp.int32)])
    sd2 = jnp.stack([src.reshape(E_PAD // B_EDGE, B_EDGE),
                     dst.reshape(E_PAD // B_EDGE, B_EDGE)], axis=1)
    zrows = jnp.zeros((R_NODES, 128), jnp.float32)

    x_pad = jnp.pad(x, ((0, N_PAD - N), (0, 0)))
    avs1 = a_src1.reshape(8, 128)
    avd1 = a_dst1.reshape(8, 128)
    hT1, asT1, adT1 = _tc_layer1(x_pad, W1, avs1, avd1)
    alpha1 = _sc_coeff(asT1, adT1, src, dst, HEADS)
    agg1 = _sc_agg(hT1, alpha1, sd2, zrows, 8, HEADS)

    b1r = b1.reshape(8, 128)
    W2r = W2.reshape(8, 128, D_OUT)
    avs2 = a_src2.reshape(2, 128)
    avd2 = a_dst2.reshape(2, 128)
    hT2, asT2, adT2 = _tc_layer2(agg1, b1r, W2r, avs2, avd2)
    alpha2 = _sc_coeff(asT2, adT2, src, dst, 1)
    agg2 = _sc_agg(hT2, alpha2, sd2, zrows, 2, 1)

    out = jnp.concatenate([agg2[0, :N, :], agg2[1, :N, :]], axis=1)
    return out + b2[None, :]


# speedup vs baseline: 1.5253x; 1.0012x over previous
import functools

import jax
import jax.numpy as jnp
from jax import lax
from jax.experimental import pallas as pl
from jax.experimental.pallas import tpu as pltpu
from jax.experimental.pallas import tpu_sc as plsc

N = 10000
D_IN = 256
HID = 256
HEADS = 4
D_OUT = 256

NC = 2
NS = 16
NW = NC * NS

N_PAD = 10240
R_NODES = N_PAD // NS
E_TOT = 160000 + N
B_EDGE = 128
E_PAD = 172032
Q_EDGE = E_PAD // NS
NB = Q_EDGE // B_EDGE
BM = 1024
NM = N_PAD // BM

def _mesh():
    return plsc.VectorSubcoreMesh(
        core_axis_name="c", subcore_axis_name="s",
        num_cores=NC, num_subcores=NS)


def _tc1_body(x_ref, w_ref, avs_ref, avd_ref, hT_ref, asT_ref, adT_ref):
    c = pl.program_id(1)
    h = jnp.dot(x_ref[...], w_ref[...], preferred_element_type=jnp.float32)
    hT_ref[0] = h
    rows = lax.broadcasted_iota(jnp.int32, (8, 128), 0)
    a_s = jnp.sum(jnp.where(rows == c, avs_ref[...], 0.0), axis=0)
    a_d = jnp.sum(jnp.where(rows == c, avd_ref[...], 0.0), axis=0)
    pas = jnp.sum(h * a_s[None, :], axis=1)
    pad = jnp.sum(h * a_d[None, :], axis=1)
    head = c // 2
    rmask = lax.broadcasted_iota(jnp.int32, (8, BM), 0) == head
    vs = jnp.where(rmask, pas[None, :], 0.0)
    vd = jnp.where(rmask, pad[None, :], 0.0)

    @pl.when(c == 0)
    def _():
        asT_ref[...] = vs[:, None, :]
        adT_ref[...] = vd[:, None, :]

    @pl.when(c != 0)
    def _():
        asT_ref[...] += vs[:, None, :]
        adT_ref[...] += vd[:, None, :]


def _tc_layer1(x_pad, W1, avs, avd):
    return pl.pallas_call(
        _tc1_body,
        grid=(NM, 8),
        in_specs=[
            pl.BlockSpec((BM, D_IN), lambda i, c: (i, 0)),
            pl.BlockSpec((D_IN, 128), lambda i, c: (0, c)),
            pl.BlockSpec((8, 128), lambda i, c: (0, 0)),
            pl.BlockSpec((8, 128), lambda i, c: (0, 0)),
        ],
        out_specs=[
            pl.BlockSpec((1, BM, 128), lambda i, c: (c, i, 0)),
            pl.BlockSpec((8, 1, BM), lambda i, c: (0, 0, i)),
            pl.BlockSpec((8, 1, BM), lambda i, c: (0, 0, i)),
        ],
        out_shape=[
            jax.ShapeDtypeStruct((8, N_PAD, 128), jnp.float32),
            jax.ShapeDtypeStruct((8, 1, N_PAD), jnp.float32),
            jax.ShapeDtypeStruct((8, 1, N_PAD), jnp.float32),
        ],
    )(x_pad, W1, avs, avd)


def _tc2_body(agg_ref, b1_ref, w_ref, avs_ref, avd_ref, hT_ref, asT_ref, adT_ref):
    kc = pl.program_id(1)
    rows = lax.broadcasted_iota(jnp.int32, (8, 128), 0)
    b1c = jnp.sum(jnp.where(rows == kc, b1_ref[...], 0.0), axis=0)
    v = agg_ref[0] + b1c[None, :]
    xb = jnp.where(v > 0, v, jnp.exp(jnp.minimum(v, 0.0)) - 1.0)
    partial = jnp.dot(xb, w_ref[0], preferred_element_type=jnp.float32)

    @pl.when(kc == 0)
    def _():
        hT_ref[0] = partial[:, :128]
        hT_ref[1] = partial[:, 128:]

    @pl.when(kc != 0)
    def _():
        hT_ref[0] += partial[:, :128]
        hT_ref[1] += partial[:, 128:]

    @pl.when(kc == 7)
    def _():
        h0 = hT_ref[0]
        h1 = hT_ref[1]
        as2 = jnp.sum(h0 * avs_ref[0][None, :], axis=1) + jnp.sum(
            h1 * avs_ref[1][None, :], axis=1)
        ad2 = jnp.sum(h0 * avd_ref[0][None, :], axis=1) + jnp.sum(
            h1 * avd_ref[1][None, :], axis=1)
        rmask = lax.broadcasted_iota(jnp.int32, (8, BM), 0) == 0
        asT_ref[...] = jnp.where(rmask, as2[None, :], 0.0)[:, None, :]
        adT_ref[...] = jnp.where(rmask, ad2[None, :], 0.0)[:, None, :]


def _tc_layer2(agg1, b1r, W2r, avs2, avd2):
    return pl.pallas_call(
        _tc2_body,
        grid=(NM, 8),
        in_specs=[
            pl.BlockSpec((1, BM, 128), lambda i, kc: (kc, i, 0)),
            pl.BlockSpec((8, 128), lambda i, kc: (0, 0)),
            pl.BlockSpec((1, 128, D_OUT), lambda i, kc: (kc, 0, 0)),
            pl.BlockSpec((2, 128), lambda i, kc: (0, 0)),
            pl.BlockSpec((2, 128), lambda i, kc: (0, 0)),
        ],
        out_specs=[
            pl.BlockSpec((2, BM, 128), lambda i, kc: (0, i, 0)),
            pl.BlockSpec((8, 1, BM), lambda i, kc: (0, 0, i)),
            pl.BlockSpec((8, 1, BM), lambda i, kc: (0, 0, i)),
        ],
        out_shape=[
            jax.ShapeDtypeStruct((2, N_PAD, 128), jnp.float32),
            jax.ShapeDtypeStruct((8, 1, N_PAD), jnp.float32),
            jax.ShapeDtypeStruct((8, 1, N_PAD), jnp.float32),
        ],
    )(agg1, b1r, W2r, avs2, avd2)


def _coeff_body(hpc, n_heads, split_phase3, asT, adT, src, dst, alphaT,
                *refs):
    as_v = refs[:hpc]
    ad_v = refs[hpc:2 * hpc]
    s_v = refs[2 * hpc:3 * hpc]
    src_v, dst_v, alpha_v, acc_v, tmp_v, slots = refs[3 * hpc:]
    cid = lax.axis_index("c")
    sid = lax.axis_index("s")
    zero16 = jnp.zeros((16,), jnp.float32)

    for h in range(hpc):
        row = cid * hpc + h if n_heads > 1 else h
        pltpu.sync_copy(asT.at[row, 0], as_v[h])
        pltpu.sync_copy(adT.at[row, 0], ad_v[h])

    def zbody(j, _):
        for h in range(hpc):
            s_v[h][pl.ds(j * 16, 16)] = zero16
        return 0
    lax.fori_loop(0, N_PAD // 16, zbody, 0)

    lanes = lax.broadcasted_iota(jnp.int32, (16,), 0)
    dnums = lax.GatherDimensionNumbers(
        offset_dims=(), collapsed_slice_dims=(0,), start_index_map=(0,))
    gmax = []
    for h in range(hpc):
        def mbody(j, m, h=h):
            return jnp.maximum(m, as_v[h][pl.ds(j * 16, 16)])
        mv = lax.fori_loop(0, N_PAD // 16, mbody,
                           jnp.full((16,), -3.0e38, jnp.float32))
        for sh in (8, 4, 2, 1):
            perm = lax.gather(mv, jnp.bitwise_xor(lanes, sh)[:, None],
                              dnums, slice_sizes=(1,),
                              mode=lax.GatherScatterMode.PROMISE_IN_BOUNDS)
            mv = jnp.maximum(mv, perm)
        gmax.append(mv)

    ebase = sid * Q_EDGE
    pltpu.sync_copy(src.at[pl.ds(ebase, Q_EDGE)], src_v)
    pltpu.sync_copy(dst.at[pl.ds(ebase, Q_EDGE)], dst_v)

    def ebody(j, _):
        si = src_v[pl.ds(j * 16, 16)]
        di = dst_v[pl.ds(j * 16, 16)]
        for h in range(hpc):
            a_s = plsc.load_gather(as_v[h], [si])
            a_d = plsc.load_gather(ad_v[h], [di])
            e = a_s + a_d
            e = jnp.where(e > 0, e, 0.2 * e)
            mt = jnp.maximum(gmax[h] + a_d, 0.0)
            ex = jnp.exp(e - mt)
            plsc.addupdate_scatter(s_v[h], [di], ex)
        return 0
    lax.fori_loop(0, Q_EDGE // 16, ebody, 0)

    for h in range(hpc):
        pltpu.sync_copy(s_v[h], slots.at[sid, h])
    plsc.subcore_barrier()
    nbase = sid * R_NODES
    pltpu.sync_copy(slots.at[0, :, pl.ds(nbase, R_NODES)], acc_v)
    for t in range(1, NS):
        pltpu.sync_copy(slots.at[t, :, pl.ds(nbase, R_NODES)], tmp_v)

        def rbody(j, _):
            for h in range(hpc):
                acc_v[h, pl.ds(j * 16, 16)] += tmp_v[h, pl.ds(j * 16, 16)]
            return 0
        lax.fori_loop(0, R_NODES // 16, rbody, 0)
    plsc.subcore_barrier()
    pltpu.sync_copy(acc_v, slots.at[0, :, pl.ds(nbase, R_NODES)])
    plsc.subcore_barrier()
    for h in range(hpc):
        pltpu.sync_copy(slots.at[0, h], s_v[h])

    if split_phase3:
        q3 = E_PAD // (2 * NS)
        eb3 = cid * (E_PAD // 2) + sid * q3
        pltpu.sync_copy(src.at[pl.ds(eb3, q3)], src_v.at[pl.ds(0, q3)])
        pltpu.sync_copy(dst.at[pl.ds(eb3, q3)], dst_v.at[pl.ds(0, q3)])
    else:
        q3 = Q_EDGE
        eb3 = ebase

    for h in range(hpc):
        def abody(j, _, h=h):
            si = src_v[pl.ds(j * 16, 16)]
            di = dst_v[pl.ds(j * 16, 16)]
            a_s = plsc.load_gather(as_v[h], [si])
            a_d = plsc.load_gather(ad_v[h], [di])
            e = a_s + a_d
            e = jnp.where(e > 0, e, 0.2 * e)
            mt = jnp.maximum(gmax[h] + a_d, 0.0)
            ex = jnp.exp(e - mt)
            sv = plsc.load_gather(s_v[h], [di])
            alpha_v[pl.ds(j * 16, 16)] = ex / (sv + 1e-16)
            return 0
        lax.fori_loop(0, q3 // 16, abody, 0)
        row = cid * hpc + h if n_heads > 1 else h
        pltpu.sync_copy(alpha_v.at[pl.ds(0, q3)],
                        alphaT.at[row, 0, pl.ds(eb3, q3)])


def _sc_coeff(asT, adT, src, dst, n_heads):
    hpc = max(n_heads // NC, 1)
    split_phase3 = n_heads == 1
    body = functools.partial(_coeff_body, hpc, n_heads, split_phase3)
    f = pl.kernel(
        body,
        out_type=jax.ShapeDtypeStruct((n_heads, 1, E_PAD), jnp.float32),
        mesh=_mesh(),
        scratch_types=[
            *([pltpu.VMEM((N_PAD,), jnp.float32)] * (3 * hpc)),
            pltpu.VMEM((Q_EDGE,), jnp.int32),
            pltpu.VMEM((Q_EDGE,), jnp.int32),
            pltpu.VMEM((Q_EDGE,), jnp.float32),
            pltpu.VMEM((hpc, R_NODES), jnp.float32),
            pltpu.VMEM((hpc, R_NODES), jnp.float32),
            pltpu.VMEM_SHARED((NS, hpc, N_PAD), jnp.float32),
        ],
        compiler_params=pltpu.CompilerParams(needs_layout_passes=False),
    )
    return f(asT, adT, src, dst)


def _agg_body(cpc, n_heads, hT, alphaT, sd2, zrows, out,
              sd_v, alpha_v, rows_v, gsem, acc):
    cid = lax.axis_index("c")
    sid = lax.axis_index("s")
    ebase = sid * Q_EDGE
    rbase = sid * NB
    nbase = sid * R_NODES

    pltpu.sync_copy(sd2.at[pl.ds(rbase, NB)], sd_v)
    for cc in range(cpc):
        g = cid * cpc + cc
        hrow = g // 2 if n_heads > 1 else 0
        pltpu.sync_copy(alphaT.at[hrow, 0, pl.ds(ebase, Q_EDGE)], alpha_v)
        pltpu.sync_copy(zrows, acc.at[pl.ds(nbase, R_NODES)])
        plsc.subcore_barrier()

        def bbody(b, _):
            pltpu.async_copy(hT.at[g].at[sd_v.at[b, 0]], rows_v,
                             gsem).wait()

            def sbody(jj, _):
                av = alpha_v[pl.ds(b * B_EDGE + jj * 16, 16)]
                for ii in range(16):
                    a = av[ii]
                    i = jj * 16 + ii
                    for r in range(8):
                        rows_v[i, pl.ds(r * 16, 16)] = (
                            rows_v[i, pl.ds(r * 16, 16)] * a)
                return 0
            lax.fori_loop(0, B_EDGE // 16, sbody, 0)
            pltpu.sync_copy(rows_v, acc.at[sd_v.at[b, 1]], add=True)
            return 0
        lax.fori_loop(0, NB, bbody, 0)
        plsc.subcore_barrier()
        pltpu.sync_copy(acc.at[pl.ds(nbase, R_NODES)],
                        out.at[g].at[pl.ds(nbase, R_NODES)])


def _sc_agg(hT, alphaT, sd2, zrows, n_chunks, n_heads):
    cpc = n_chunks // NC
    body = functools.partial(_agg_body, cpc, n_heads)
    f = pl.kernel(
        body,
        out_type=jax.ShapeDtypeStruct((n_chunks, N_PAD, 128), jnp.float32),
        mesh=_mesh(),
        scratch_types=[
            pltpu.VMEM((NB, 2, B_EDGE), jnp.int32),
            pltpu.VMEM((Q_EDGE,), jnp.float32),
            pltpu.VMEM((B_EDGE, 128), jnp.float32),
            pltpu.SemaphoreType.DMA,
            pltpu.VMEM_SHARED((N_PAD, 128), jnp.float32),
        ],
    )
    return f(hT, alphaT, sd2, zrows)


def kernel(x, edge_index, W1, a_src1, a_dst1, b1, W2, a_src2, a_dst2, b2):
    idx = edge_index.astype(jnp.int32)
    loop = jnp.arange(N, dtype=jnp.int32)
    src = jnp.concatenate(
        [idx[0], loop, jnp.zeros((E_PAD - E_TOT,), jnp.int32)])
    dst = jnp.concatenate(
        [idx[1], loop, jnp.full((E_PAD - E_TOT,), N, jnp.int32)])
    sd2 = jnp.stack([src.reshape(E_PAD // B_EDGE, B_EDGE),
                     dst.reshape(E_PAD // B_EDGE, B_EDGE)], axis=1)
    zrows = jnp.zeros((R_NODES, 128), jnp.float32)

    x_pad = jnp.pad(x, ((0, N_PAD - N), (0, 0)))
    avs1 = a_src1.reshape(8, 128)
    avd1 = a_dst1.reshape(8, 128)
    hT1, asT1, adT1 = _tc_layer1(x_pad, W1, avs1, avd1)
    alpha1 = _sc_coeff(asT1, adT1, src, dst, HEADS)
    agg1 = _sc_agg(hT1, alpha1, sd2, zrows, 8, HEADS)

    b1r = b1.reshape(8, 128)
    W2r = W2.reshape(8, 128, D_OUT)
    avs2 = a_src2.reshape(2, 128)
    avd2 = a_dst2.reshape(2, 128)
    hT2, asT2, adT2 = _tc_layer2(agg1, b1r, W2r, avs2, avd2)
    alpha2 = _sc_coeff(asT2, adT2, src, dst, 1)
    agg2 = _sc_agg(hT2, alpha2, sd2, zrows, 2, 1)

    out = jnp.concatenate([agg2[0, :N, :], agg2[1, :N, :]], axis=1)
    return out + b2[None, :]
